# Initial kernel scaffold; baseline (speedup 1.0000x reference)
import jax
import jax.numpy as jnp
from jax.experimental import pallas as pl


def kernel(x, Wg, Wu, Wd):
    raise NotImplementedError("write your pallas kernel here")


# speedup vs baseline: 37.3794x; 37.3794x over previous
import functools

import jax
import jax.numpy as jnp
from jax.experimental import pallas as pl
from jax.experimental.pallas import tpu as pltpu

HIDDEN = 1024
INTER = 2816
K_ACTIVE = 704
ROW_TILE = 256


def _fused_kernel(x_ref, wg_ref, wu_ref, wd_ref, out_ref):
    x = x_ref[...]

    nt = (((1,), (1,)), ((), ()))
    g = jax.lax.dot_general(x, wg_ref[...], nt, preferred_element_type=jnp.float32)
    u = jax.lax.dot_general(x, wu_ref[...], nt, preferred_element_type=jnp.float32)
    z = (g * jax.nn.sigmoid(g)) * u

    ab = jax.lax.bitcast_convert_type(z, jnp.int32) & jnp.int32(0x7FFFFFFF)

    def body(i, res):
        cand = res | (jnp.int32(1) << (jnp.int32(30) - i))
        cnt = jnp.sum((ab >= cand).astype(jnp.int32), axis=1, keepdims=True)
        return jnp.where(cnt >= K_ACTIVE, cand, res)

    res0 = jnp.zeros((z.shape[0], 1), jnp.int32)
    res = jax.lax.fori_loop(0, 31, body, res0)

    zm = jnp.where(ab >= res, z, 0.0).astype(jnp.bfloat16)
    out_ref[...] = jax.lax.dot_general(
        zm, wd_ref[...], nt, preferred_element_type=jnp.float32
    )


@jax.jit
def kernel(x, Wg, Wu, Wd):
    B, S, H = x.shape
    rows = B * S
    xb = x.reshape(rows, H).astype(jnp.bfloat16)

    out = pl.pallas_call(
        _fused_kernel,
        grid=(rows // ROW_TILE,),
        in_specs=[
            pl.BlockSpec((ROW_TILE, HIDDEN), lambda i: (i, 0)),
            pl.BlockSpec((INTER, HIDDEN), lambda i: (0, 0)),
            pl.BlockSpec((INTER, HIDDEN), lambda i: (0, 0)),
            pl.BlockSpec((HIDDEN, INTER), lambda i: (0, 0)),
        ],
        out_specs=pl.BlockSpec((ROW_TILE, HIDDEN), lambda i: (i, 0)),
        out_shape=jax.ShapeDtypeStruct((rows, HIDDEN), jnp.float32),
        compiler_params=pltpu.CompilerParams(
            dimension_semantics=("arbitrary",),
        ),
    )(
        xb,
        Wg.astype(jnp.bfloat16),
        Wu.astype(jnp.bfloat16),
        Wd.astype(jnp.bfloat16),
    )
    return out.reshape(B, S, H)


# speedup vs baseline: 39.3163x; 1.0518x over previous
import functools

import jax
import jax.numpy as jnp
from jax.experimental import pallas as pl
from jax.experimental.pallas import tpu as pltpu

HIDDEN = 1024
INTER = 2816
K_ACTIVE = 704
ROW_TILE = 256


def _fused_kernel(x_ref, wg_ref, wu_ref, wd_ref, out_ref):
    x = x_ref[...]

    nt = (((1,), (1,)), ((), ()))
    g = jax.lax.dot_general(x, wg_ref[...], nt, preferred_element_type=jnp.float32)
    u = jax.lax.dot_general(x, wu_ref[...], nt, preferred_element_type=jnp.float32)
    z = (g * jax.nn.sigmoid(g)) * u

    ab = jax.lax.bitcast_convert_type(z, jnp.int32) & jnp.int32(0x7FFFFFFF)

    def body(i, res):
        cand = res | (jnp.int32(1) << (jnp.int32(30) - i))
        ones = jnp.where(ab >= cand, 1.0, 0.0)
        cnt = jnp.sum(ones, axis=1, keepdims=True)
        return jnp.where(cnt >= float(K_ACTIVE), cand, res)

    res0 = jnp.zeros((z.shape[0], 1), jnp.int32)
    res = jax.lax.fori_loop(0, 31, body, res0)

    zm = jnp.where(ab >= res, z, 0.0).astype(jnp.bfloat16)
    out_ref[...] = jax.lax.dot_general(
        zm, wd_ref[...], nt, preferred_element_type=jnp.float32
    )


@jax.jit
def kernel(x, Wg, Wu, Wd):
    B, S, H = x.shape
    rows = B * S
    xb = x.reshape(rows, H).astype(jnp.bfloat16)

    out = pl.pallas_call(
        _fused_kernel,
        grid=(rows // ROW_TILE,),
        in_specs=[
            pl.BlockSpec((ROW_TILE, HIDDEN), lambda i: (i, 0)),
            pl.BlockSpec((INTER, HIDDEN), lambda i: (0, 0)),
            pl.BlockSpec((INTER, HIDDEN), lambda i: (0, 0)),
            pl.BlockSpec((HIDDEN, INTER), lambda i: (0, 0)),
        ],
        out_specs=pl.BlockSpec((ROW_TILE, HIDDEN), lambda i: (i, 0)),
        out_shape=jax.ShapeDtypeStruct((rows, HIDDEN), jnp.float32),
        compiler_params=pltpu.CompilerParams(
            dimension_semantics=("arbitrary",),
        ),
    )(
        xb,
        Wg.astype(jnp.bfloat16),
        Wu.astype(jnp.bfloat16),
        Wd.astype(jnp.bfloat16),
    )
    return out.reshape(B, S, H)


# speedup vs baseline: 45.0123x; 1.1449x over previous
import functools

import jax
import jax.numpy as jnp
from jax.experimental import pallas as pl
from jax.experimental.pallas import tpu as pltpu

HIDDEN = 1024
INTER = 2816
K_ACTIVE = 704
ROW_TILE = 256


def _fused_kernel(x_ref, wg_ref, wu_ref, wd_ref, out_ref):
    x = x_ref[...]

    nt = (((1,), (1,)), ((), ()))
    g = jax.lax.dot_general(x, wg_ref[...], nt, preferred_element_type=jnp.float32)
    u = jax.lax.dot_general(x, wu_ref[...], nt, preferred_element_type=jnp.float32)
    z = (g * jax.nn.sigmoid(g)) * u

    az = jnp.abs(z)

    def body(i, res):
        cand = res | (jnp.int32(1) << (jnp.int32(30) - i))
        candf = jax.lax.bitcast_convert_type(cand, jnp.float32)
        ones = jnp.where(az >= candf, 1.0, 0.0)
        cnt = jnp.sum(ones, axis=1, keepdims=True)
        return jnp.where(cnt >= float(K_ACTIVE), cand, res)

    res0 = jnp.zeros((z.shape[0], 1), jnp.int32)
    res = jax.lax.fori_loop(0, 25, body, res0)

    thresh = jax.lax.bitcast_convert_type(res, jnp.float32)
    zm = jnp.where(az >= thresh, z, 0.0).astype(jnp.bfloat16)
    out_ref[...] = jax.lax.dot_general(
        zm, wd_ref[...], nt, preferred_element_type=jnp.float32
    )


@jax.jit
def kernel(x, Wg, Wu, Wd):
    B, S, H = x.shape
    rows = B * S
    xb = x.reshape(rows, H).astype(jnp.bfloat16)

    out = pl.pallas_call(
        _fused_kernel,
        grid=(rows // ROW_TILE,),
        in_specs=[
            pl.BlockSpec((ROW_TILE, HIDDEN), lambda i: (i, 0)),
            pl.BlockSpec((INTER, HIDDEN), lambda i: (0, 0)),
            pl.BlockSpec((INTER, HIDDEN), lambda i: (0, 0)),
            pl.BlockSpec((HIDDEN, INTER), lambda i: (0, 0)),
        ],
        out_specs=pl.BlockSpec((ROW_TILE, HIDDEN), lambda i: (i, 0)),
        out_shape=jax.ShapeDtypeStruct((rows, HIDDEN), jnp.float32),
        compiler_params=pltpu.CompilerParams(
            dimension_semantics=("arbitrary",),
        ),
    )(
        xb,
        Wg.astype(jnp.bfloat16),
        Wu.astype(jnp.bfloat16),
        Wd.astype(jnp.bfloat16),
    )
    return out.reshape(B, S, H)


# speedup vs baseline: 52.8128x; 1.1733x over previous
import functools

import jax
import jax.numpy as jnp
from jax.experimental import pallas as pl
from jax.experimental.pallas import tpu as pltpu

HIDDEN = 1024
INTER = 2816
K_ACTIVE = 704
ROW_TILE = 256


def _fused_kernel(x_ref, wg_ref, wu_ref, wd_ref, out_ref, az_ref):
    x = x_ref[...].astype(jnp.bfloat16)

    nt = (((1,), (1,)), ((), ()))
    g = jax.lax.dot_general(x, wg_ref[...], nt, preferred_element_type=jnp.float32)
    u = jax.lax.dot_general(x, wu_ref[...], nt, preferred_element_type=jnp.float32)
    z = (g * jax.nn.sigmoid(g)) * u
    az_ref[...] = jnp.abs(z)

    def body(i, res):
        cand = res | (jnp.int32(1) << (jnp.int32(30) - i))
        candf = jax.lax.bitcast_convert_type(cand, jnp.float32)
        ones = jnp.where(az_ref[...] >= candf, 1.0, 0.0)
        cnt = jnp.sum(ones, axis=1, keepdims=True)
        return jnp.where(cnt >= float(K_ACTIVE), cand, res)

    res0 = jnp.zeros((ROW_TILE, 1), jnp.int32)
    res = jax.lax.fori_loop(0, 25, body, res0)

    thresh = jax.lax.bitcast_convert_type(res, jnp.float32)
    zm = jnp.where(az_ref[...] >= thresh, z, 0.0).astype(jnp.bfloat16)
    out_ref[...] = jax.lax.dot_general(
        zm, wd_ref[...], nt, preferred_element_type=jnp.float32
    )


@jax.jit
def kernel(x, Wg, Wu, Wd):
    B, S, H = x.shape
    rows = B * S
    xf = x.reshape(rows, H)

    out = pl.pallas_call(
        _fused_kernel,
        grid=(rows // ROW_TILE,),
        in_specs=[
            pl.BlockSpec((ROW_TILE, HIDDEN), lambda i: (i, 0)),
            pl.BlockSpec((INTER, HIDDEN), lambda i: (0, 0)),
            pl.BlockSpec((INTER, HIDDEN), lambda i: (0, 0)),
            pl.BlockSpec((HIDDEN, INTER), lambda i: (0, 0)),
        ],
        out_specs=pl.BlockSpec((ROW_TILE, HIDDEN), lambda i: (i, 0)),
        out_shape=jax.ShapeDtypeStruct((rows, HIDDEN), jnp.float32),
        scratch_shapes=[pltpu.VMEM((ROW_TILE, INTER), jnp.float32)],
        compiler_params=pltpu.CompilerParams(
            dimension_semantics=("arbitrary",),
        ),
    )(
        xf,
        Wg.astype(jnp.bfloat16),
        Wu.astype(jnp.bfloat16),
        Wd.astype(jnp.bfloat16),
    )
    return out.reshape(B, S, H)
